# baseline (speedup 1.0000x reference)
import functools

import jax
import jax.numpy as jnp
from jax import lax
from jax.experimental import pallas as pl
from jax.experimental.pallas import tpu as pltpu

_K = 40

_HI = lax.Precision.HIGHEST


def _proj_body(vert_ref, wcat_ref, brow_ref, bcol_ref,
               prop_ref, dims_ref, t_ref, *, P, S):
    v = vert_ref[...]
    w = wcat_ref[...]
    y = jnp.dot(v, w, preferred_element_type=jnp.float32) + brow_ref[...]
    prop_ref[...] = y[:, 0:P]
    dims_ref[...] = y[:, P:P + S]
    yT = lax.dot_general(w, v, (((0,), (1,)), ((), ())),
                         preferred_element_type=jnp.float32) + bcol_ref[...]
    dimsT = yT[P:P + S, :]
    norms = jnp.sum(dimsT * dimsT, axis=0, keepdims=True)
    t_ref[...] = jnp.concatenate([dimsT, norms, yT[0:P, :]], axis=0)


def _main_body(vert_ref, t_ref, pa_ref, dblk_ref,
               wv_ref, wmx_ref, wmn_ref, bo_ref, out_ref, *, N, P, R):
    dimsT = t_ref[0:4, :]
    norms = t_ref[4:5, :]
    dims_blk = dblk_ref[...]
    ab = lax.dot_general(dims_blk, dimsT, (((1,), (0,)), ((), ())),
                         preferred_element_type=jnp.float32)
    dotA = jnp.sum(dims_blk * dims_blk, axis=1, keepdims=True)
    D = jnp.abs(dotA + norms - 2.0 * ab)

    bits = lax.bitcast_convert_type(D, jnp.int32)

    def _count(mask):
        return jnp.sum(mask.astype(jnp.int32), axis=1, keepdims=True)

    def vbody(i, prefix):
        cand = prefix | (jnp.int32(1) << (jnp.int32(30) - i))
        return jnp.where(_count(bits < cand) >= _K, prefix, cand)

    v40 = lax.fori_loop(0, 31, vbody, jnp.zeros((R, 1), jnp.int32))

    lt = bits < v40
    tie = bits == v40
    need = _K - _count(lt)
    iota = lax.broadcasted_iota(jnp.int32, (R, N), 1)

    def ibody(i, tp):
        cand = tp | (jnp.int32(1) << (jnp.int32(11) - i))
        return jnp.where(_count(tie & (iota < cand)) >= need, tp, cand)

    all_ties_fit = jnp.all(need == _count(tie))
    tsel = lax.cond(
        all_ties_fit,
        lambda: jnp.full((R, 1), N, jnp.int32),
        lambda: lax.fori_loop(0, 12, ibody, jnp.zeros((R, 1), jnp.int32)))
    sel = lt | (tie & (iota <= tsel))

    mbits = jnp.min(bits, axis=1, keepdims=True)
    mpos = jnp.min(jnp.where(bits == mbits, iota, N), axis=1, keepdims=True)
    sel = sel & (iota != mpos)

    w = jnp.exp(-jnp.square(D * 10.0))
    wz = jnp.where(sel, w, 0.0)

    propA = pa_ref[...]
    ssum = lax.dot_general(wz, propA, (((1,), (0,)), ((), ())),
                           preferred_element_type=jnp.float32)
    mean = ssum * (1.0 / (_K - 1))

    neg = jnp.float32(-jnp.inf)
    nnA = (propA >= 0.0).astype(jnp.float32)
    C2 = jnp.dot(wz, nnA, preferred_element_type=jnp.float32)

    def _fast():
        return jnp.concatenate(
            [jnp.max(wz * t_ref[5 + p:6 + p, :], axis=1, keepdims=True)
             for p in range(P)], axis=1)

    def _exact():
        return jnp.concatenate(
            [jnp.max(jnp.where(sel, wz * t_ref[5 + p:6 + p, :], neg),
                     axis=1, keepdims=True)
             for p in range(P)], axis=1)

    mx = lax.cond(jnp.min(C2) > 0.0, _fast, _exact)

    pre = (jnp.dot(vert_ref[...], wv_ref[...],
                   preferred_element_type=jnp.float32)
           + jnp.dot(mx, wmx_ref[...], preferred_element_type=jnp.float32)
           + jnp.dot(mean, wmn_ref[...], preferred_element_type=jnp.float32)
           + bo_ref[...])
    out_ref[...] = jnp.tanh(pre)


def _build(B, N, F, P, S, O, interpret=False):
    RP = min(N, 1024)
    R = min(N, 512)

    proj = pl.pallas_call(
        functools.partial(_proj_body, P=P, S=S),
        grid=(B, N // RP),
        in_specs=[
            pl.BlockSpec((None, RP, F), lambda b, i: (b, i, 0)),
            pl.BlockSpec((F, P + S), lambda b, i: (0, 0)),
            pl.BlockSpec((1, P + S), lambda b, i: (0, 0)),
            pl.BlockSpec((P + S, 1), lambda b, i: (0, 0)),
        ],
        out_specs=[
            pl.BlockSpec((None, RP, P), lambda b, i: (b, i, 0)),
            pl.BlockSpec((None, RP, S), lambda b, i: (b, i, 0)),
            pl.BlockSpec((None, S + 1 + P, RP), lambda b, i: (b, 0, i)),
        ],
        out_shape=[
            jax.ShapeDtypeStruct((B, N, P), jnp.float32),
            jax.ShapeDtypeStruct((B, N, S), jnp.float32),
            jax.ShapeDtypeStruct((B, S + 1 + P, N), jnp.float32),
        ],
        compiler_params=pltpu.CompilerParams(
            dimension_semantics=("parallel", "parallel")),
        interpret=interpret,
    )

    main = pl.pallas_call(
        functools.partial(_main_body, N=N, P=P, R=R),
        grid=(B, N // R),
        in_specs=[
            pl.BlockSpec((None, R, F), lambda b, i: (b, i, 0)),
            pl.BlockSpec((None, S + 1 + P, N), lambda b, i: (b, 0, 0)),
            pl.BlockSpec((None, N, P), lambda b, i: (b, 0, 0)),
            pl.BlockSpec((None, R, S), lambda b, i: (b, i, 0)),
            pl.BlockSpec((F, O), lambda b, i: (0, 0)),
            pl.BlockSpec((P, O), lambda b, i: (0, 0)),
            pl.BlockSpec((P, O), lambda b, i: (0, 0)),
            pl.BlockSpec((1, O), lambda b, i: (0, 0)),
        ],
        out_specs=pl.BlockSpec((None, R, O), lambda b, i: (b, i, 0)),
        out_shape=jax.ShapeDtypeStruct((B, N, O), jnp.float32),
        compiler_params=pltpu.CompilerParams(
            dimension_semantics=("parallel", "parallel")),
        interpret=interpret,
    )
    return proj, main


def _run(vertices_in, W_prop, b_prop, W_dim, b_dim, W_out, b_out,
         interpret=False):
    B, N, F = vertices_in.shape
    P = W_prop.shape[1]
    S = W_dim.shape[1]
    O = W_out.shape[1]
    proj, main = _build(B, N, F, P, S, O, interpret=interpret)
    wcat = jnp.concatenate([W_prop, W_dim], axis=1)
    bcat = jnp.concatenate([b_prop, b_dim], axis=0)
    prop, dims, t = proj(vertices_in, wcat,
                         bcat.reshape(1, P + S), bcat.reshape(P + S, 1))
    return main(vertices_in, t, prop, dims,
                W_out[0:F], W_out[F:F + P], W_out[F + P:F + 2 * P],
                b_out.reshape(1, O))


def kernel(vertices_in, W_prop, b_prop, W_dim, b_dim, W_out, b_out):
    return _run(vertices_in, W_prop, b_prop, W_dim, b_dim, W_out, b_out)


# speedup vs baseline: 1.0584x; 1.0584x over previous
import functools

import jax
import jax.numpy as jnp
from jax import lax
from jax.experimental import pallas as pl
from jax.experimental.pallas import tpu as pltpu

_K = 40

_HI = lax.Precision.HIGHEST


def _proj_body(vert_ref, wcat_ref, brow_ref, bcol_ref,
               prop_ref, dims_ref, t_ref, *, P, S):
    v = vert_ref[...]
    w = wcat_ref[...]
    y = jnp.dot(v, w, preferred_element_type=jnp.float32) + brow_ref[...]
    prop_ref[...] = y[:, 0:P]
    dims_ref[...] = y[:, P:P + S]
    yT = lax.dot_general(w, v, (((0,), (1,)), ((), ())),
                         preferred_element_type=jnp.float32) + bcol_ref[...]
    dimsT = yT[P:P + S, :]
    norms = jnp.sum(dimsT * dimsT, axis=0, keepdims=True)
    t_ref[...] = jnp.concatenate([dimsT, norms, yT[0:P, :]], axis=0)


def _main_body(vert_ref, t_ref, pa_ref, dblk_ref,
               wv_ref, wmx_ref, wmn_ref, bo_ref, out_ref, *, N, P, R):
    dimsT = t_ref[0:4, :]
    norms = t_ref[4:5, :]
    dims_blk = dblk_ref[...]
    ab = lax.dot_general(dims_blk, dimsT, (((1,), (0,)), ((), ())),
                         preferred_element_type=jnp.float32)
    dotA = jnp.sum(dims_blk * dims_blk, axis=1, keepdims=True)
    D = jnp.abs(dotA + norms - 2.0 * ab)

    bits = lax.bitcast_convert_type(D, jnp.int32)

    def _count(mask):
        return jnp.sum(mask.astype(jnp.int32), axis=1, keepdims=True)

    def vbody(i, prefix):
        cand = prefix | (jnp.int32(1) << (jnp.int32(30) - i))
        return jnp.where(_count(bits < cand) >= _K, prefix, cand)

    v40 = jnp.zeros((R, 1), jnp.int32)
    for i in range(31):
        v40 = vbody(i, v40)

    lt = bits < v40
    tie = bits == v40
    need = _K - _count(lt)
    iota = lax.broadcasted_iota(jnp.int32, (R, N), 1)

    def ibody(i, tp):
        cand = tp | (jnp.int32(1) << (jnp.int32(11) - i))
        return jnp.where(_count(tie & (iota < cand)) >= need, tp, cand)

    all_ties_fit = jnp.all(need == _count(tie))
    tsel = lax.cond(
        all_ties_fit,
        lambda: jnp.full((R, 1), N, jnp.int32),
        lambda: lax.fori_loop(0, 12, ibody, jnp.zeros((R, 1), jnp.int32)))
    sel = lt | (tie & (iota <= tsel))

    mbits = jnp.min(bits, axis=1, keepdims=True)
    mpos = jnp.min(jnp.where(bits == mbits, iota, N), axis=1, keepdims=True)
    sel = sel & (iota != mpos)

    L = -jnp.square(D * 10.0)
    wsel = jnp.where(sel, jnp.exp(L), 0.0)

    propA = pa_ref[...]
    ssum = lax.dot_general(wsel, propA, (((1,), (0,)), ((), ())),
                           preferred_element_type=jnp.float32)
    mean = ssum * (1.0 / (_K - 1))

    neg = jnp.float32(-jnp.inf)
    M1 = jnp.where(sel, L, neg)
    cols = []
    for p in range(P):
        row = t_ref[5 + p:6 + p, :]
        ofs = jnp.where(row > 0.0, jnp.log(row), neg)
        g = jnp.max(M1 + ofs, axis=1, keepdims=True)
        mx_p = lax.cond(
            jnp.any(g == neg),
            lambda row=row: jnp.max(jnp.where(sel, wsel * row, neg),
                                    axis=1, keepdims=True),
            lambda g=g: jnp.exp(g))
        cols.append(mx_p)
    mx = jnp.concatenate(cols, axis=1)

    pre = (jnp.dot(vert_ref[...], wv_ref[...],
                   preferred_element_type=jnp.float32)
           + jnp.dot(mx, wmx_ref[...], preferred_element_type=jnp.float32)
           + jnp.dot(mean, wmn_ref[...], preferred_element_type=jnp.float32)
           + bo_ref[...])
    out_ref[...] = jnp.tanh(pre)


def _build(B, N, F, P, S, O, interpret=False):
    RP = min(N, 1024)
    R = min(N, 512)

    proj = pl.pallas_call(
        functools.partial(_proj_body, P=P, S=S),
        grid=(B, N // RP),
        in_specs=[
            pl.BlockSpec((None, RP, F), lambda b, i: (b, i, 0)),
            pl.BlockSpec((F, P + S), lambda b, i: (0, 0)),
            pl.BlockSpec((1, P + S), lambda b, i: (0, 0)),
            pl.BlockSpec((P + S, 1), lambda b, i: (0, 0)),
        ],
        out_specs=[
            pl.BlockSpec((None, RP, P), lambda b, i: (b, i, 0)),
            pl.BlockSpec((None, RP, S), lambda b, i: (b, i, 0)),
            pl.BlockSpec((None, S + 1 + P, RP), lambda b, i: (b, 0, i)),
        ],
        out_shape=[
            jax.ShapeDtypeStruct((B, N, P), jnp.float32),
            jax.ShapeDtypeStruct((B, N, S), jnp.float32),
            jax.ShapeDtypeStruct((B, S + 1 + P, N), jnp.float32),
        ],
        compiler_params=pltpu.CompilerParams(
            dimension_semantics=("parallel", "parallel")),
        interpret=interpret,
    )

    main = pl.pallas_call(
        functools.partial(_main_body, N=N, P=P, R=R),
        grid=(B, N // R),
        in_specs=[
            pl.BlockSpec((None, R, F), lambda b, i: (b, i, 0)),
            pl.BlockSpec((None, S + 1 + P, N), lambda b, i: (b, 0, 0)),
            pl.BlockSpec((None, N, P), lambda b, i: (b, 0, 0)),
            pl.BlockSpec((None, R, S), lambda b, i: (b, i, 0)),
            pl.BlockSpec((F, O), lambda b, i: (0, 0)),
            pl.BlockSpec((P, O), lambda b, i: (0, 0)),
            pl.BlockSpec((P, O), lambda b, i: (0, 0)),
            pl.BlockSpec((1, O), lambda b, i: (0, 0)),
        ],
        out_specs=pl.BlockSpec((None, R, O), lambda b, i: (b, i, 0)),
        out_shape=jax.ShapeDtypeStruct((B, N, O), jnp.float32),
        compiler_params=pltpu.CompilerParams(
            dimension_semantics=("parallel", "parallel")),
        interpret=interpret,
    )
    return proj, main


def _run(vertices_in, W_prop, b_prop, W_dim, b_dim, W_out, b_out,
         interpret=False):
    B, N, F = vertices_in.shape
    P = W_prop.shape[1]
    S = W_dim.shape[1]
    O = W_out.shape[1]
    proj, main = _build(B, N, F, P, S, O, interpret=interpret)
    wcat = jnp.concatenate([W_prop, W_dim], axis=1)
    bcat = jnp.concatenate([b_prop, b_dim], axis=0)
    prop, dims, t = proj(vertices_in, wcat,
                         bcat.reshape(1, P + S), bcat.reshape(P + S, 1))
    return main(vertices_in, t, prop, dims,
                W_out[0:F], W_out[F:F + P], W_out[F + P:F + 2 * P],
                b_out.reshape(1, O))


def kernel(vertices_in, W_prop, b_prop, W_dim, b_dim, W_out, b_out):
    return _run(vertices_in, W_prop, b_prop, W_dim, b_dim, W_out, b_out)


# speedup vs baseline: 1.3163x; 1.2437x over previous
import functools

import jax
import jax.numpy as jnp
from jax import lax
from jax.experimental import pallas as pl
from jax.experimental.pallas import tpu as pltpu

_K = 40

_HI = lax.Precision.HIGHEST


def _proj_body(vert_ref, wcat_ref, brow_ref, bcol_ref,
               prop_ref, dims_ref, t_ref, *, P, S):
    v = vert_ref[...]
    w = wcat_ref[...]
    y = jnp.dot(v, w, preferred_element_type=jnp.float32) + brow_ref[...]
    prop_ref[...] = y[:, 0:P]
    dims_ref[...] = y[:, P:P + S]
    yT = lax.dot_general(w, v, (((0,), (1,)), ((), ())),
                         preferred_element_type=jnp.float32) + bcol_ref[...]
    dimsT = yT[P:P + S, :]
    norms = jnp.sum(dimsT * dimsT, axis=0, keepdims=True)
    t_ref[...] = jnp.concatenate([dimsT, norms, yT[0:P, :]], axis=0)


def _main_body(vert_ref, t_ref, pa_ref, dblk_ref,
               wv_ref, wmx_ref, wmn_ref, bo_ref, out_ref, *, N, P, R):
    dimsT = t_ref[0:4, :]
    norms = t_ref[4:5, :]
    dims_blk = dblk_ref[...]
    ab = lax.dot_general(dims_blk, dimsT, (((1,), (0,)), ((), ())),
                         preferred_element_type=jnp.float32)
    dotA = jnp.sum(dims_blk * dims_blk, axis=1, keepdims=True)
    D = jnp.abs(dotA + norms - 2.0 * ab)

    bits = lax.bitcast_convert_type(D, jnp.int32)

    def _count(mask):
        return jnp.sum(mask.astype(jnp.int32), axis=1, keepdims=True)

    def vbody(i, prefix):
        cand = prefix | (jnp.int32(1) << (jnp.int32(30) - i))
        return jnp.where(_count(bits < cand) >= _K, prefix, cand)

    v40 = lax.fori_loop(0, 31, vbody, jnp.zeros((R, 1), jnp.int32))

    lt = bits < v40
    tie = bits == v40
    need = _K - _count(lt)
    iota = lax.broadcasted_iota(jnp.int32, (R, N), 1)

    def ibody(i, tp):
        cand = tp | (jnp.int32(1) << (jnp.int32(11) - i))
        return jnp.where(_count(tie & (iota < cand)) >= need, tp, cand)

    all_ties_fit = jnp.all(need == _count(tie))
    tsel = lax.cond(
        all_ties_fit,
        lambda: jnp.full((R, 1), N, jnp.int32),
        lambda: lax.fori_loop(0, 12, ibody, jnp.zeros((R, 1), jnp.int32)))
    sel = lt | (tie & (iota <= tsel))

    mbits = jnp.min(bits, axis=1, keepdims=True)
    mpos = jnp.min(jnp.where(bits == mbits, iota, N), axis=1, keepdims=True)
    sel = sel & (iota != mpos)

    L = -jnp.square(D * 10.0)
    wsel = jnp.where(sel, jnp.exp(L), 0.0)

    propA = pa_ref[...]
    ssum = lax.dot_general(wsel, propA, (((1,), (0,)), ((), ())),
                           preferred_element_type=jnp.float32)
    mean = ssum * (1.0 / (_K - 1))

    neg = jnp.float32(-jnp.inf)
    M1 = jnp.where(sel, L, neg)
    cols = []
    for p in range(P):
        row = t_ref[5 + p:6 + p, :]
        ofs = jnp.where(row > 0.0, jnp.log(row), neg)
        g = jnp.max(M1 + ofs, axis=1, keepdims=True)
        mx_p = lax.cond(
            jnp.any(g == neg),
            lambda row=row: jnp.max(jnp.where(sel, wsel * row, neg),
                                    axis=1, keepdims=True),
            lambda g=g: jnp.exp(g))
        cols.append(mx_p)
    mx = jnp.concatenate(cols, axis=1)

    pre = (jnp.dot(vert_ref[...], wv_ref[...],
                   preferred_element_type=jnp.float32)
           + jnp.dot(mx, wmx_ref[...], preferred_element_type=jnp.float32)
           + jnp.dot(mean, wmn_ref[...], preferred_element_type=jnp.float32)
           + bo_ref[...])
    out_ref[...] = jnp.tanh(pre)


def _build(B, N, F, P, S, O, interpret=False):
    RP = min(N, 1024)
    R = min(N, 256)

    proj = pl.pallas_call(
        functools.partial(_proj_body, P=P, S=S),
        grid=(B, N // RP),
        in_specs=[
            pl.BlockSpec((None, RP, F), lambda b, i: (b, i, 0)),
            pl.BlockSpec((F, P + S), lambda b, i: (0, 0)),
            pl.BlockSpec((1, P + S), lambda b, i: (0, 0)),
            pl.BlockSpec((P + S, 1), lambda b, i: (0, 0)),
        ],
        out_specs=[
            pl.BlockSpec((None, RP, P), lambda b, i: (b, i, 0)),
            pl.BlockSpec((None, RP, S), lambda b, i: (b, i, 0)),
            pl.BlockSpec((None, S + 1 + P, RP), lambda b, i: (b, 0, i)),
        ],
        out_shape=[
            jax.ShapeDtypeStruct((B, N, P), jnp.float32),
            jax.ShapeDtypeStruct((B, N, S), jnp.float32),
            jax.ShapeDtypeStruct((B, S + 1 + P, N), jnp.float32),
        ],
        compiler_params=pltpu.CompilerParams(
            dimension_semantics=("parallel", "parallel")),
        interpret=interpret,
    )

    main = pl.pallas_call(
        functools.partial(_main_body, N=N, P=P, R=R),
        grid=(B, N // R),
        in_specs=[
            pl.BlockSpec((None, R, F), lambda b, i: (b, i, 0)),
            pl.BlockSpec((None, S + 1 + P, N), lambda b, i: (b, 0, 0)),
            pl.BlockSpec((None, N, P), lambda b, i: (b, 0, 0)),
            pl.BlockSpec((None, R, S), lambda b, i: (b, i, 0)),
            pl.BlockSpec((F, O), lambda b, i: (0, 0)),
            pl.BlockSpec((P, O), lambda b, i: (0, 0)),
            pl.BlockSpec((P, O), lambda b, i: (0, 0)),
            pl.BlockSpec((1, O), lambda b, i: (0, 0)),
        ],
        out_specs=pl.BlockSpec((None, R, O), lambda b, i: (b, i, 0)),
        out_shape=jax.ShapeDtypeStruct((B, N, O), jnp.float32),
        compiler_params=pltpu.CompilerParams(
            dimension_semantics=("parallel", "parallel")),
        interpret=interpret,
    )
    return proj, main


def _run(vertices_in, W_prop, b_prop, W_dim, b_dim, W_out, b_out,
         interpret=False):
    B, N, F = vertices_in.shape
    P = W_prop.shape[1]
    S = W_dim.shape[1]
    O = W_out.shape[1]
    proj, main = _build(B, N, F, P, S, O, interpret=interpret)
    wcat = jnp.concatenate([W_prop, W_dim], axis=1)
    bcat = jnp.concatenate([b_prop, b_dim], axis=0)
    prop, dims, t = proj(vertices_in, wcat,
                         bcat.reshape(1, P + S), bcat.reshape(P + S, 1))
    return main(vertices_in, t, prop, dims,
                W_out[0:F], W_out[F:F + P], W_out[F + P:F + 2 * P],
                b_out.reshape(1, O))


def kernel(vertices_in, W_prop, b_prop, W_dim, b_dim, W_out, b_out):
    return _run(vertices_in, W_prop, b_prop, W_dim, b_dim, W_out, b_out)
